# Initial kernel scaffold; baseline (speedup 1.0000x reference)
import jax
import jax.numpy as jnp
from jax.experimental import pallas as pl


def kernel(x, edge_index_follows, edge_index_likes, edge_index_views, W_follows, W_likes, W_views, F_follows, F_likes, F_views, ln_gamma, ln_beta):
    raise NotImplementedError("write your pallas kernel here")


# speedup vs baseline: 3.0741x; 3.0741x over previous
import functools

import jax
import jax.numpy as jnp
from jax import lax
from jax.experimental import pallas as pl
from jax.experimental.pallas import tpu as pltpu
from jax.experimental.pallas import tpu_sc as plsc

_N = 10000
_D = 128
_OUT = 128
_NC = 2
_NS = 16
_NW = _NC * _NS
_K = 128
_ZR = 128
_RSP = 10240
_DUMMY = _N


def _dense_body(x_ref, w_ref, o_ref):
    big = jnp.dot(x_ref[...], w_ref[...], preferred_element_type=jnp.float32)
    for et in range(3):
        base = et * 3 * _OUT
        msg = big[:, base:base + _OUT]
        gam = big[:, base + _OUT:base + 2 * _OUT]
        bet = big[:, base + 2 * _OUT:base + 3 * _OUT]
        o_ref[et] = jnp.maximum(gam * msg + bet, 0.0)


def _dense(x, wcat):
    br = 1000
    return pl.pallas_call(
        _dense_body,
        grid=(_N // br,),
        in_specs=[
            pl.BlockSpec((br, _D), lambda i: (i, 0)),
            pl.BlockSpec((_D, 9 * _OUT), lambda i: (0, 0)),
        ],
        out_specs=pl.BlockSpec((3, br, _OUT), lambda i: (0, i, 0)),
        out_shape=jax.ShapeDtypeStruct((3, _N, _OUT), jnp.float32),
    )(x, wcat)


def _scatter(m2, srcs, dsts):
    nb = srcs.shape[1]
    mesh = plsc.VectorSubcoreMesh(core_axis_name="c", subcore_axis_name="s")

    @functools.partial(
        pl.kernel,
        mesh=mesh,
        out_type=jax.ShapeDtypeStruct((_NC, _RSP, _OUT), jnp.float32),
        scratch_types=[
            pltpu.VMEM((_K,), jnp.int32),
            pltpu.VMEM((_K,), jnp.int32),
            pltpu.VMEM((_K, _OUT), jnp.float32),
            pltpu.VMEM((_ZR, _OUT), jnp.float32),
            pltpu.VMEM_SHARED((_RSP, _OUT), jnp.float32),
            pltpu.SemaphoreType.DMA,
        ],
    )
    def body(m_hbm, srcs_hbm, dsts_hbm, out_hbm, sidx, didx, rows, zbuf, h_sp, sem):
        cid = lax.axis_index("c")
        sid = lax.axis_index("s")

        def zb(t, c):
            i = t // (_OUT // 16)
            j = t % (_OUT // 16)
            zbuf[i, pl.ds(j * 16, 16)] = jnp.zeros((16,), jnp.float32)
            return c

        lax.fori_loop(0, _ZR * (_OUT // 16), zb, 0)

        rows_per_sub = _RSP // _NS

        def zh(t, c):
            pltpu.sync_copy(
                zbuf, h_sp.at[pl.ds(sid * rows_per_sub + t * _ZR, _ZR)])
            return c

        lax.fori_loop(0, rows_per_sub // _ZR, zh, 0)
        plsc.subcore_barrier()

        w = sid * _NC + cid

        def eb(bi, c):
            pltpu.sync_copy(srcs_hbm.at[w, bi], sidx)
            pltpu.sync_copy(dsts_hbm.at[w, bi], didx)
            pltpu.async_copy(m_hbm.at[sidx], rows, sem).wait()
            pltpu.sync_copy(rows, h_sp.at[didx], add=True)
            return c

        lax.fori_loop(0, nb, eb, 0)
        plsc.subcore_barrier()

        rpc = _RSP // _NS
        pltpu.sync_copy(h_sp.at[pl.ds(sid * rpc, rpc)],
                        out_hbm.at[cid, pl.ds(sid * rpc, rpc)])

    return body(m2, srcs, dsts)


def _ln_body(p_ref, g_ref, b_ref, o_ref):
    h = p_ref[0] + p_ref[1]
    mean = jnp.mean(h, axis=-1, keepdims=True)
    cen = h - mean
    var = jnp.mean(cen * cen, axis=-1, keepdims=True)
    o_ref[...] = (cen / jnp.sqrt(var + 1e-5)) * g_ref[...] + b_ref[...]


def _ln(partials, g, b):
    br = 1000
    return pl.pallas_call(
        _ln_body,
        grid=(_N // br,),
        in_specs=[
            pl.BlockSpec((_NC, br, _OUT), lambda i: (0, i, 0)),

            pl.BlockSpec((1, _OUT), lambda i: (0, 0)),
            pl.BlockSpec((1, _OUT), lambda i: (0, 0)),
        ],
        out_specs=pl.BlockSpec((br, _OUT), lambda i: (i, 0)),
        out_shape=jax.ShapeDtypeStruct((_N, _OUT), jnp.float32),
    )(partials, g, b)


def kernel(x, edge_index_follows, edge_index_likes, edge_index_views,
           W_follows, W_likes, W_views,
           F_follows, F_likes, F_views,
           ln_gamma, ln_beta):
    wcat = jnp.concatenate(
        [W_follows, F_follows, W_likes, F_likes, W_views, F_views], axis=1)
    m = _dense(x, wcat)
    m2 = m.reshape(3 * _N, _OUT)

    src = jnp.concatenate([
        edge_index_follows[0],
        edge_index_likes[0] + _N,
        edge_index_views[0] + 2 * _N,
    ]).astype(jnp.int32)
    dst = jnp.concatenate([
        edge_index_follows[1],
        edge_index_likes[1],
        edge_index_views[1],
    ]).astype(jnp.int32)

    e3 = src.shape[0]
    nb = -(-e3 // (_NW * _K))
    pad = _NW * _K * nb - e3
    if pad:
        src = jnp.concatenate([src, jnp.zeros((pad,), jnp.int32)])
        dst = jnp.concatenate([dst, jnp.full((pad,), _DUMMY, jnp.int32)])
    srcs = src.reshape(_NW, nb, _K)
    dsts = dst.reshape(_NW, nb, _K)

    partials = _scatter(m2, srcs, dsts)
    return _ln(partials, ln_gamma.reshape(1, -1), ln_beta.reshape(1, -1))


# speedup vs baseline: 5.0272x; 1.6353x over previous
import functools

import jax
import jax.numpy as jnp
from jax import lax
from jax.experimental import pallas as pl
from jax.experimental.pallas import tpu as pltpu
from jax.experimental.pallas import tpu_sc as plsc

_N = 10000
_D = 128
_OUT = 128
_NC = 2
_NS = 16
_NW = _NC * _NS
_K = 112
_RSP = 10112
_DUMMY = _N


def _dense_body(x_ref, w_ref, o_ref):
    big = jnp.dot(x_ref[...], w_ref[...], preferred_element_type=jnp.float32)
    for et in range(3):
        base = et * 3 * _OUT
        msg = big[:, base:base + _OUT]
        gam = big[:, base + _OUT:base + 2 * _OUT]
        bet = big[:, base + 2 * _OUT:base + 3 * _OUT]
        o_ref[et] = jnp.maximum(gam * msg + bet, 0.0)


def _dense(x, wcat):
    br = 1000
    return pl.pallas_call(
        _dense_body,
        grid=(_N // br,),
        in_specs=[
            pl.BlockSpec((br, _D), lambda i: (i, 0)),
            pl.BlockSpec((_D, 9 * _OUT), lambda i: (0, 0)),
        ],
        out_specs=pl.BlockSpec((3, br, _OUT), lambda i: (0, i, 0)),
        out_shape=jax.ShapeDtypeStruct((3, _N, _OUT), jnp.float32),
    )(x, wcat)


def _scatter(m2, srcs, dsts):
    npair = srcs.shape[1]
    mesh = plsc.VectorSubcoreMesh(core_axis_name="c", subcore_axis_name="s")

    @functools.partial(
        pl.kernel,
        mesh=mesh,
        out_type=jax.ShapeDtypeStruct((_NC, _RSP, _OUT), jnp.float32),
        scratch_types=[
            pltpu.VMEM((2, _K), jnp.int32),
            pltpu.VMEM((2, _K), jnp.int32),
            pltpu.VMEM((2, _K), jnp.int32),
            pltpu.VMEM((2, _K), jnp.int32),
            pltpu.VMEM((_K, _OUT), jnp.float32),
            pltpu.VMEM((_K, _OUT), jnp.float32),
            pltpu.VMEM_SHARED((_RSP, _OUT), jnp.float32),
            pltpu.SemaphoreType.DMA,
            pltpu.SemaphoreType.DMA,
            pltpu.SemaphoreType.DMA,
            pltpu.SemaphoreType.DMA,
        ],
    )
    def body(m_hbm, srcs_hbm, dsts_hbm, out_hbm,
             siA, diA, siB, diB, rows0, rows1, h_sp,
             g0, g1, isemA, isemB):
        cid = lax.axis_index("c")
        sid = lax.axis_index("s")
        w = sid * _NC + cid

        pltpu.async_copy(srcs_hbm.at[w, 0], siA, isemA)
        pltpu.async_copy(dsts_hbm.at[w, 0], diA, isemA)
        pltpu.async_copy(srcs_hbm.at[w, 1], siB, isemB)
        pltpu.async_copy(dsts_hbm.at[w, 1], diB, isemB)

        def zb(t, c):
            i = t // (_OUT // 16)
            j = t % (_OUT // 16)
            rows1[i, pl.ds(j * 16, 16)] = jnp.zeros((16,), jnp.float32)
            return c

        lax.fori_loop(0, _K * (_OUT // 16), zb, 0)

        rows_per_sub = _RSP // _NS
        base = sid * rows_per_sub
        nfull = rows_per_sub // _K
        for t in range(nfull):
            pltpu.sync_copy(rows1, h_sp.at[pl.ds(base + t * _K, _K)])
        rem = rows_per_sub - nfull * _K
        if rem:
            pltpu.sync_copy(rows1.at[pl.ds(0, rem)],
                            h_sp.at[pl.ds(base + nfull * _K, rem)])
        plsc.subcore_barrier()

        pltpu.make_async_copy(srcs_hbm.at[w, 0], siA, isemA).wait()
        pltpu.make_async_copy(dsts_hbm.at[w, 0], diA, isemA).wait()
        pltpu.async_copy(m_hbm.at[siA.at[0]], rows0, g0)

        def do_pair(p, S_si, S_di, T_si, T_di, isemS, isemT):
            pltpu.async_copy(m_hbm.at[S_si.at[1]], rows1, g1)
            pltpu.make_async_copy(m_hbm.at[S_si.at[0]], rows0, g0).wait()
            pltpu.sync_copy(rows0, h_sp.at[S_di.at[0]], add=True)

            @pl.when(p + 1 < npair)
            def _():
                pltpu.make_async_copy(srcs_hbm.at[w, p + 1], T_si, isemT).wait()
                pltpu.make_async_copy(dsts_hbm.at[w, p + 1], T_di, isemT).wait()

            pltpu.make_async_copy(m_hbm.at[S_si.at[1]], rows1, g1).wait()
            pltpu.sync_copy(rows1, h_sp.at[S_di.at[1]], add=True)

            @pl.when(p + 2 < npair)
            def _():
                pltpu.async_copy(srcs_hbm.at[w, p + 2], S_si, isemS)
                pltpu.async_copy(dsts_hbm.at[w, p + 2], S_di, isemS)

            @pl.when(p + 1 < npair)
            def _():
                pltpu.async_copy(m_hbm.at[T_si.at[0]], rows0, g0)

        def tbody(t, c):
            do_pair(2 * t, siA, diA, siB, diB, isemA, isemB)
            do_pair(2 * t + 1, siB, diB, siA, diA, isemB, isemA)
            return c

        lax.fori_loop(0, npair // 2, tbody, 0)
        plsc.subcore_barrier()

        pltpu.sync_copy(h_sp.at[pl.ds(base, rows_per_sub)],
                        out_hbm.at[cid, pl.ds(base, rows_per_sub)])

    return body(m2, srcs, dsts)


def _ln_body(p_ref, g_ref, b_ref, o_ref):
    h = p_ref[0] + p_ref[1]
    mean = jnp.mean(h, axis=-1, keepdims=True)
    cen = h - mean
    var = jnp.mean(cen * cen, axis=-1, keepdims=True)
    o_ref[...] = (cen / jnp.sqrt(var + 1e-5)) * g_ref[...] + b_ref[...]


def _ln(partials, g, b):
    br = 1000
    return pl.pallas_call(
        _ln_body,
        grid=(_N // br,),
        in_specs=[
            pl.BlockSpec((_NC, br, _OUT), lambda i: (0, i, 0)),

            pl.BlockSpec((1, _OUT), lambda i: (0, 0)),
            pl.BlockSpec((1, _OUT), lambda i: (0, 0)),
        ],
        out_specs=pl.BlockSpec((br, _OUT), lambda i: (i, 0)),
        out_shape=jax.ShapeDtypeStruct((_N, _OUT), jnp.float32),
    )(partials, g, b)


def kernel(x, edge_index_follows, edge_index_likes, edge_index_views,
           W_follows, W_likes, W_views,
           F_follows, F_likes, F_views,
           ln_gamma, ln_beta):
    wcat = jnp.concatenate(
        [W_follows, F_follows, W_likes, F_likes, W_views, F_views], axis=1)
    m = _dense(x, wcat)
    m2 = m.reshape(3 * _N, _OUT)

    src = jnp.concatenate([
        edge_index_follows[0],
        edge_index_likes[0] + _N,
        edge_index_views[0] + 2 * _N,
    ]).astype(jnp.int32)
    dst = jnp.concatenate([
        edge_index_follows[1],
        edge_index_likes[1],
        edge_index_views[1],
    ]).astype(jnp.int32)

    e3 = src.shape[0]
    epw = -(-e3 // _NW)
    npair = -(-epw // (2 * _K))
    npair += npair % 2
    epw_pad = npair * 2 * _K
    pad = _NW * epw_pad - e3
    if pad:
        src = jnp.concatenate([src, jnp.zeros((pad,), jnp.int32)])
        dst = jnp.concatenate([dst, jnp.full((pad,), _DUMMY, jnp.int32)])
    srcs = src.reshape(_NW, npair, 2, _K)
    dsts = dst.reshape(_NW, npair, 2, _K)

    partials = _scatter(m2, srcs, dsts)
    return _ln(partials, ln_gamma.reshape(1, -1), ln_beta.reshape(1, -1))
